# Initial kernel scaffold; baseline (speedup 1.0000x reference)
import functools

import jax
import jax.numpy as jnp
from jax import lax
from jax.experimental import pallas as pl
from jax.experimental.pallas import tpu as pltpu
from jax.experimental.pallas import tpu_sc as plsc

N = 10000
E = 320000
C = 128

NC = 2
NS = 16
CHUNK = 80
CPT = E // (NC * NS * CHUNK)
RPT = N // NS
DEG_W = 16

_mesh = plsc.VectorSubcoreMesh(core_axis_name="c", subcore_axis_name="s")


def _zero_vmem(ref, rows, width):
    z = jnp.zeros((16,), jnp.float32)

    @pl.loop(0, rows)
    def _(r):
        @pl.loop(0, width, step=16)
        def _(cc):
            ref[r, pl.ds(cc, 16)] = z


def _deg_body(dst_hbm, out_hbm, acc_sp, idx_v, ones_v, zb_v, sem):
    c = lax.axis_index("c")
    s = lax.axis_index("s")

    _zero_vmem(zb_v, RPT, DEG_W)
    one = jnp.ones((16,), jnp.float32)

    @pl.loop(0, CHUNK)
    def _(r):
        ones_v[r, pl.ds(0, 16)] = one

    pltpu.async_copy(zb_v, acc_sp.at[pl.ds(s * RPT, RPT)], sem).wait()
    plsc.subcore_barrier()

    row0 = (c * NS + s) * CPT

    @pl.loop(0, CPT)
    def _(i):
        pltpu.async_copy(dst_hbm.at[row0 + i], idx_v, sem).wait()
        pltpu.sync_copy(ones_v, acc_sp.at[idx_v], add=True)

    plsc.subcore_barrier()
    pltpu.async_copy(acc_sp.at[pl.ds(s * RPT, RPT)],
                     out_hbm.at[c, pl.ds(s * RPT, RPT)], sem).wait()


@jax.jit
def _deg(dst2d):
    k = pl.kernel(
        _deg_body,
        out_type=jax.ShapeDtypeStruct((NC, N, DEG_W), jnp.float32),
        mesh=_mesh,
        scratch_types=[
            pltpu.VMEM_SHARED((N, DEG_W), jnp.float32),
            pltpu.VMEM((CHUNK,), jnp.int32),
            pltpu.VMEM((CHUNK, DEG_W), jnp.float32),
            pltpu.VMEM((RPT, DEG_W), jnp.float32),
            pltpu.SemaphoreType.DMA,
        ],
    )
    return k(dst2d)


def _agg_body(m_hbm, src_hbm, dst_hbm, out_hbm, acc_sp,
              sidx_v, didx_v, rows_a, rows_b, zb_v, sem_a, sem_b, sem):
    c = lax.axis_index("c")
    s = lax.axis_index("s")

    _zero_vmem(zb_v, RPT // 5, C)

    @pl.loop(0, 5)
    def _(j):
        pltpu.async_copy(zb_v, acc_sp.at[pl.ds(s * RPT + j * (RPT // 5),
                                               RPT // 5)], sem).wait()

    row0 = (c * NS + s) * CPT
    pltpu.async_copy(src_hbm.at[pl.ds(row0, CPT)], sidx_v, sem).wait()
    pltpu.async_copy(dst_hbm.at[pl.ds(row0, CPT)], didx_v, sem).wait()
    plsc.subcore_barrier()

    pltpu.async_copy(m_hbm.at[sidx_v.at[0]], rows_a, sem_a)

    @pl.loop(0, CPT, step=2)
    def _(i):
        pltpu.make_async_copy(m_hbm.at[sidx_v.at[0]], rows_a, sem_a).wait()

        @pl.when(i + 1 < CPT)
        def _():
            pltpu.async_copy(m_hbm.at[sidx_v.at[i + 1]], rows_b, sem_b)

        pltpu.sync_copy(rows_a, acc_sp.at[didx_v.at[i]], add=True)

        @pl.when(i + 1 < CPT)
        def _():
            pltpu.make_async_copy(m_hbm.at[sidx_v.at[0]], rows_b, sem_b).wait()

            @pl.when(i + 2 < CPT)
            def _():
                pltpu.async_copy(m_hbm.at[sidx_v.at[i + 2]], rows_a, sem_a)

            pltpu.sync_copy(rows_b, acc_sp.at[didx_v.at[i + 1]], add=True)

    plsc.subcore_barrier()
    pltpu.async_copy(acc_sp.at[pl.ds(s * RPT, RPT)],
                     out_hbm.at[c, pl.ds(s * RPT, RPT)], sem).wait()


@jax.jit
def _agg(m, src2d, dst2d):
    k = pl.kernel(
        _agg_body,
        out_type=jax.ShapeDtypeStruct((NC, N, C), jnp.float32),
        mesh=_mesh,
        scratch_types=[
            pltpu.VMEM_SHARED((N, C), jnp.float32),
            pltpu.VMEM((CPT, CHUNK), jnp.int32),
            pltpu.VMEM((CPT, CHUNK), jnp.int32),
            pltpu.VMEM((CHUNK, C), jnp.float32),
            pltpu.VMEM((CHUNK, C), jnp.float32),
            pltpu.VMEM((RPT // 5, C), jnp.float32),
            pltpu.SemaphoreType.DMA,
            pltpu.SemaphoreType.DMA,
            pltpu.SemaphoreType.DMA,
        ],
    )
    return k(m, src2d, dst2d)


BLK = 1000


def _dis_of(dp):
    deg = dp[0, :, 0] + dp[1, :, 0] + 1.0
    return lax.rsqrt(deg)


def _pre_body(x_ref, w_ref, dp_ref, o_ref):
    dis = _dis_of(dp_ref[...])
    h = jnp.dot(x_ref[...], w_ref[...], preferred_element_type=jnp.float32)
    o_ref[...] = h * dis[:, None]


def _mid_body(agg_ref, m_ref, dp_ref, b_ref, w_ref, o_ref):
    dis = _dis_of(dp_ref[...])
    t = (agg_ref[0] + agg_ref[1] + m_ref[...]) * dis[:, None] + b_ref[...]
    h = jnp.where(t > 0, t, jnp.expm1(t))
    o_ref[...] = jnp.dot(h, w_ref[...],
                         preferred_element_type=jnp.float32) * dis[:, None]


def _post_body(agg_ref, m_ref, dp_ref, b_ref, o_ref):
    dis = _dis_of(dp_ref[...])
    t = (agg_ref[0] + agg_ref[1] + m_ref[...]) * dis[:, None] + b_ref[...]
    o_ref[...] = jnp.where(t > 0, t, jnp.expm1(t))


_row_blk = pl.BlockSpec((BLK, C), lambda i: (i, 0))
_full_w = pl.BlockSpec((C, C), lambda i: (0, 0))
_dp_blk = pl.BlockSpec((NC, BLK, DEG_W), lambda i: (0, i, 0))
_agg_blk = pl.BlockSpec((NC, BLK, C), lambda i: (0, i, 0))
_bias_blk = pl.BlockSpec((1, C), lambda i: (0, 0))
_out_t = jax.ShapeDtypeStruct((N, C), jnp.float32)


@jax.jit
def _pre(x, W1, dp):
    return pl.pallas_call(
        _pre_body, grid=(N // BLK,),
        in_specs=[_row_blk, _full_w, _dp_blk],
        out_specs=_row_blk, out_shape=_out_t,
    )(x, W1, dp)


@jax.jit
def _mid(agg, m, dp, b, W2):
    return pl.pallas_call(
        _mid_body, grid=(N // BLK,),
        in_specs=[_agg_blk, _row_blk, _dp_blk, _bias_blk, _full_w],
        out_specs=_row_blk, out_shape=_out_t,
    )(agg, m, dp, b, W2)


@jax.jit
def _post(agg, m, dp, b):
    return pl.pallas_call(
        _post_body, grid=(N // BLK,),
        in_specs=[_agg_blk, _row_blk, _dp_blk, _bias_blk],
        out_specs=_row_blk, out_shape=_out_t,
    )(agg, m, dp, b)


def kernel(x, edge_index, W1, b1, W2, b2):
    src2d = edge_index[0].astype(jnp.int32).reshape(E // CHUNK, CHUNK)
    dst2d = edge_index[1].astype(jnp.int32).reshape(E // CHUNK, CHUNK)
    b1r = b1.reshape(1, C)
    b2r = b2.reshape(1, C)

    dp = _deg(dst2d)
    m1 = _pre(x, W1, dp)
    agg1 = _agg(m1, src2d, dst2d)
    m2 = _mid(agg1, m1, dp, b1r, W2)
    agg2 = _agg(m2, src2d, dst2d)
    return _post(agg2, m2, dp, b2r)


# speedup vs baseline: 19.2518x; 19.2518x over previous
import functools

import jax
import jax.numpy as jnp
from jax import lax
from jax.experimental import pallas as pl
from jax.experimental.pallas import tpu as pltpu
from jax.experimental.pallas import tpu_sc as plsc

N = 10000
E = 320000
C = 128

NC = 2
NS = 16
CHUNK = 50
CPT = E // (NC * NS * CHUNK)
IB = 40
NPAD = 10240
RPT = NPAD // NS
DEG_W = 16
ZB = 32

_mesh = plsc.VectorSubcoreMesh(core_axis_name="c", subcore_axis_name="s")


def _fill_vmem(ref, rows, width, value):
    v = jnp.full((16,), value, jnp.float32)

    @pl.loop(0, rows)
    def _(r):
        @pl.loop(0, width, step=16)
        def _(cc):
            ref[r, pl.ds(cc, 16)] = v


def _deg_body(dst_hbm, out_hbm, acc_sp, didx_v, ones_v, zb_v, sem):
    c = lax.axis_index("c")
    s = lax.axis_index("s")

    _fill_vmem(zb_v, RPT, DEG_W, 0.0)
    _fill_vmem(ones_v, CHUNK, DEG_W, 1.0)

    pltpu.async_copy(zb_v, acc_sp.at[pl.ds(s * RPT, RPT)], sem).wait()

    row0 = (c * NS + s) * CPT
    pltpu.async_copy(dst_hbm.at[pl.ds(row0, CPT)], didx_v, sem).wait()
    plsc.subcore_barrier()

    @pl.loop(0, CPT)
    def _(i):
        pltpu.sync_copy(ones_v, acc_sp.at[didx_v.at[i]], add=True)

    plsc.subcore_barrier()
    pltpu.async_copy(acc_sp.at[pl.ds(s * RPT, RPT)],
                     out_hbm.at[c, pl.ds(s * RPT, RPT)], sem).wait()


@jax.jit
def _deg(dst2d):
    k = pl.kernel(
        _deg_body,
        out_type=jax.ShapeDtypeStruct((NC, NPAD, DEG_W), jnp.float32),
        mesh=_mesh,
        scratch_types=[
            pltpu.VMEM_SHARED((NPAD, DEG_W), jnp.float32),
            pltpu.VMEM((CPT, CHUNK), jnp.int32),
            pltpu.VMEM((CHUNK, DEG_W), jnp.float32),
            pltpu.VMEM((RPT, DEG_W), jnp.float32),
            pltpu.SemaphoreType.DMA,
        ],
    )
    return k(dst2d)


def _agg_body(m_hbm, src_hbm, dst_hbm, out_hbm, acc_sp,
              sidx_v, didx_v, rows_a, rows_b, zb_v, sem_a, sem_b, sem):
    c = lax.axis_index("c")
    s = lax.axis_index("s")

    _fill_vmem(zb_v, ZB, C, 0.0)

    @pl.loop(0, RPT // ZB)
    def _(j):
        pltpu.async_copy(zb_v, acc_sp.at[pl.ds(s * RPT + j * ZB, ZB)],
                         sem).wait()

    row0 = (c * NS + s) * CPT
    plsc.subcore_barrier()

    @pl.loop(0, CPT // IB)
    def _(b):
        pltpu.async_copy(src_hbm.at[pl.ds(row0 + b * IB, IB)], sidx_v,
                         sem).wait()
        pltpu.async_copy(dst_hbm.at[pl.ds(row0 + b * IB, IB)], didx_v,
                         sem).wait()
        pltpu.async_copy(m_hbm.at[sidx_v.at[0]], rows_a, sem_a)

        @pl.loop(0, IB, step=2)
        def _(i):
            pltpu.make_async_copy(m_hbm.at[sidx_v.at[0]], rows_a, sem_a).wait()
            pltpu.async_copy(m_hbm.at[sidx_v.at[i + 1]], rows_b, sem_b)
            pltpu.sync_copy(rows_a, acc_sp.at[didx_v.at[i]], add=True)

            pltpu.make_async_copy(m_hbm.at[sidx_v.at[0]], rows_b, sem_b).wait()

            @pl.when(i + 2 < IB)
            def _():
                pltpu.async_copy(m_hbm.at[sidx_v.at[i + 2]], rows_a, sem_a)

            pltpu.sync_copy(rows_b, acc_sp.at[didx_v.at[i + 1]], add=True)

    plsc.subcore_barrier()
    pltpu.async_copy(acc_sp.at[pl.ds(s * RPT, RPT)],
                     out_hbm.at[c, pl.ds(s * RPT, RPT)], sem).wait()


@jax.jit
def _agg(m, src2d, dst2d):
    k = pl.kernel(
        _agg_body,
        out_type=jax.ShapeDtypeStruct((NC, NPAD, C), jnp.float32),
        mesh=_mesh,
        scratch_types=[
            pltpu.VMEM_SHARED((NPAD, C), jnp.float32),
            pltpu.VMEM((IB, CHUNK), jnp.int32),
            pltpu.VMEM((IB, CHUNK), jnp.int32),
            pltpu.VMEM((CHUNK, C), jnp.float32),
            pltpu.VMEM((CHUNK, C), jnp.float32),
            pltpu.VMEM((ZB, C), jnp.float32),
            pltpu.SemaphoreType.DMA,
            pltpu.SemaphoreType.DMA,
            pltpu.SemaphoreType.DMA,
        ],
    )
    return k(m, src2d, dst2d)


BLK = 1000


def _dis_of(dp):
    deg = dp[0, :, 0] + dp[1, :, 0] + 1.0
    return lax.rsqrt(deg)


def _pre_body(x_ref, w_ref, dp_ref, o_ref):
    dis = _dis_of(dp_ref[...])
    h = jnp.dot(x_ref[...], w_ref[...], preferred_element_type=jnp.float32)
    o_ref[...] = h * dis[:, None]


def _mid_body(agg_ref, m_ref, dp_ref, b_ref, w_ref, o_ref):
    dis = _dis_of(dp_ref[...])
    t = (agg_ref[0] + agg_ref[1] + m_ref[...]) * dis[:, None] + b_ref[...]
    h = jnp.where(t > 0, t, jnp.exp(jnp.minimum(t, 0.0)) - 1.0)
    o_ref[...] = jnp.dot(h, w_ref[...],
                         preferred_element_type=jnp.float32) * dis[:, None]


def _post_body(agg_ref, m_ref, dp_ref, b_ref, o_ref):
    dis = _dis_of(dp_ref[...])
    t = (agg_ref[0] + agg_ref[1] + m_ref[...]) * dis[:, None] + b_ref[...]
    o_ref[...] = jnp.where(t > 0, t, jnp.exp(jnp.minimum(t, 0.0)) - 1.0)


_row_blk = pl.BlockSpec((BLK, C), lambda i: (i, 0))
_full_w = pl.BlockSpec((C, C), lambda i: (0, 0))
_dp_blk = pl.BlockSpec((NC, BLK, DEG_W), lambda i: (0, i, 0))
_agg_blk = pl.BlockSpec((NC, BLK, C), lambda i: (0, i, 0))
_bias_blk = pl.BlockSpec((1, C), lambda i: (0, 0))
_out_t = jax.ShapeDtypeStruct((N, C), jnp.float32)


@jax.jit
def _pre(x, W1, dp):
    return pl.pallas_call(
        _pre_body, grid=(N // BLK,),
        in_specs=[_row_blk, _full_w, _dp_blk],
        out_specs=_row_blk, out_shape=_out_t,
    )(x, W1, dp)


@jax.jit
def _mid(agg, m, dp, b, W2):
    return pl.pallas_call(
        _mid_body, grid=(N // BLK,),
        in_specs=[_agg_blk, _row_blk, _dp_blk, _bias_blk, _full_w],
        out_specs=_row_blk, out_shape=_out_t,
    )(agg, m, dp, b, W2)


@jax.jit
def _post(agg, m, dp, b):
    return pl.pallas_call(
        _post_body, grid=(N // BLK,),
        in_specs=[_agg_blk, _row_blk, _dp_blk, _bias_blk],
        out_specs=_row_blk, out_shape=_out_t,
    )(agg, m, dp, b)


def kernel(x, edge_index, W1, b1, W2, b2):
    src2d = edge_index[0].astype(jnp.int32).reshape(E // CHUNK, CHUNK)
    dst2d = edge_index[1].astype(jnp.int32).reshape(E // CHUNK, CHUNK)
    b1r = b1.reshape(1, C)
    b2r = b2.reshape(1, C)

    dp = _deg(dst2d)
    m1 = _pre(x, W1, dp)
    agg1 = _agg(m1, src2d, dst2d)
    m2 = _mid(agg1, m1, dp, b1r, W2)
    agg2 = _agg(m2, src2d, dst2d)
    return _post(agg2, m2, dp, b2r)


# speedup vs baseline: 26.1580x; 1.3587x over previous
import functools

import jax
import jax.numpy as jnp
from jax import lax
from jax.experimental import pallas as pl
from jax.experimental.pallas import tpu as pltpu
from jax.experimental.pallas import tpu_sc as plsc

N = 10000
E = 320000
C = 128

NC = 2
NS = 16
CHUNK = 125
CPT = E // (NC * NS * CHUNK)
IB = 8
NPAD = 10240
RPT = NPAD // NS
DEG_W = 16
ZB = 32

_mesh = plsc.VectorSubcoreMesh(core_axis_name="c", subcore_axis_name="s")


def _fill_vmem(ref, rows, width, value):
    v = jnp.full((16,), value, jnp.float32)

    @pl.loop(0, rows)
    def _(r):
        @pl.loop(0, width, step=16)
        def _(cc):
            ref[r, pl.ds(cc, 16)] = v


def _deg_body(dst_hbm, out_hbm, acc_sp, didx_v, ones_v, zb_v, sem):
    c = lax.axis_index("c")
    s = lax.axis_index("s")

    _fill_vmem(zb_v, RPT, DEG_W, 0.0)
    _fill_vmem(ones_v, CHUNK, DEG_W, 1.0)

    pltpu.async_copy(zb_v, acc_sp.at[pl.ds(s * RPT, RPT)], sem).wait()

    row0 = (c * NS + s) * CPT
    pltpu.async_copy(dst_hbm.at[pl.ds(row0, CPT)], didx_v, sem).wait()
    plsc.subcore_barrier()

    @pl.loop(0, CPT)
    def _(i):
        pltpu.sync_copy(ones_v, acc_sp.at[didx_v.at[i]], add=True)

    plsc.subcore_barrier()
    pltpu.async_copy(acc_sp.at[pl.ds(s * RPT, RPT)],
                     out_hbm.at[c, pl.ds(s * RPT, RPT)], sem).wait()


@jax.jit
def _deg(dst2d):
    k = pl.kernel(
        _deg_body,
        out_type=jax.ShapeDtypeStruct((NC, NPAD, DEG_W), jnp.float32),
        mesh=_mesh,
        scratch_types=[
            pltpu.VMEM_SHARED((NPAD, DEG_W), jnp.float32),
            pltpu.VMEM((CPT, CHUNK), jnp.int32),
            pltpu.VMEM((CHUNK, DEG_W), jnp.float32),
            pltpu.VMEM((RPT, DEG_W), jnp.float32),
            pltpu.SemaphoreType.DMA,
        ],
    )
    return k(dst2d)


def _agg_body(m_hbm, src_hbm, dst_hbm, out_hbm, acc_sp,
              sidx_v, didx_v, rows_a, rows_b, zb_v, sem_a, sem_b, sem):
    c = lax.axis_index("c")
    s = lax.axis_index("s")

    _fill_vmem(zb_v, ZB, C, 0.0)

    @pl.loop(0, RPT // ZB)
    def _(j):
        pltpu.async_copy(zb_v, acc_sp.at[pl.ds(s * RPT + j * ZB, ZB)],
                         sem).wait()

    row0 = (c * NS + s) * CPT
    plsc.subcore_barrier()

    @pl.loop(0, CPT // IB)
    def _(b):
        pltpu.async_copy(src_hbm.at[pl.ds(row0 + b * IB, IB)], sidx_v,
                         sem).wait()
        pltpu.async_copy(dst_hbm.at[pl.ds(row0 + b * IB, IB)], didx_v,
                         sem).wait()
        pltpu.async_copy(m_hbm.at[sidx_v.at[0]], rows_a, sem_a)

        @pl.loop(0, IB, step=2)
        def _(i):
            pltpu.make_async_copy(m_hbm.at[sidx_v.at[0]], rows_a, sem_a).wait()
            pltpu.async_copy(m_hbm.at[sidx_v.at[i + 1]], rows_b, sem_b)
            pltpu.sync_copy(rows_a, acc_sp.at[didx_v.at[i]], add=True)

            pltpu.make_async_copy(m_hbm.at[sidx_v.at[0]], rows_b, sem_b).wait()

            @pl.when(i + 2 < IB)
            def _():
                pltpu.async_copy(m_hbm.at[sidx_v.at[i + 2]], rows_a, sem_a)

            pltpu.sync_copy(rows_b, acc_sp.at[didx_v.at[i + 1]], add=True)

    plsc.subcore_barrier()
    pltpu.async_copy(acc_sp.at[pl.ds(s * RPT, RPT)],
                     out_hbm.at[c, pl.ds(s * RPT, RPT)], sem).wait()


@jax.jit
def _agg(m, src2d, dst2d):
    k = pl.kernel(
        _agg_body,
        out_type=jax.ShapeDtypeStruct((NC, NPAD, C), jnp.float32),
        mesh=_mesh,
        scratch_types=[
            pltpu.VMEM_SHARED((NPAD, C), jnp.float32),
            pltpu.VMEM((IB, CHUNK), jnp.int32),
            pltpu.VMEM((IB, CHUNK), jnp.int32),
            pltpu.VMEM((CHUNK, C), jnp.float32),
            pltpu.VMEM((CHUNK, C), jnp.float32),
            pltpu.VMEM((ZB, C), jnp.float32),
            pltpu.SemaphoreType.DMA,
            pltpu.SemaphoreType.DMA,
            pltpu.SemaphoreType.DMA,
        ],
    )
    return k(m, src2d, dst2d)


BLK = 1000


def _dis_of(dp):
    deg = dp[0, :, 0] + dp[1, :, 0] + 1.0
    return lax.rsqrt(deg)


def _pre_body(x_ref, w_ref, dp_ref, o_ref):
    dis = _dis_of(dp_ref[...])
    h = jnp.dot(x_ref[...], w_ref[...], preferred_element_type=jnp.float32)
    o_ref[...] = h * dis[:, None]


def _mid_body(agg_ref, m_ref, dp_ref, b_ref, w_ref, o_ref):
    dis = _dis_of(dp_ref[...])
    t = (agg_ref[0] + agg_ref[1] + m_ref[...]) * dis[:, None] + b_ref[...]
    h = jnp.where(t > 0, t, jnp.exp(jnp.minimum(t, 0.0)) - 1.0)
    o_ref[...] = jnp.dot(h, w_ref[...],
                         preferred_element_type=jnp.float32) * dis[:, None]


def _post_body(agg_ref, m_ref, dp_ref, b_ref, o_ref):
    dis = _dis_of(dp_ref[...])
    t = (agg_ref[0] + agg_ref[1] + m_ref[...]) * dis[:, None] + b_ref[...]
    o_ref[...] = jnp.where(t > 0, t, jnp.exp(jnp.minimum(t, 0.0)) - 1.0)


_row_blk = pl.BlockSpec((BLK, C), lambda i: (i, 0))
_full_w = pl.BlockSpec((C, C), lambda i: (0, 0))
_dp_blk = pl.BlockSpec((NC, BLK, DEG_W), lambda i: (0, i, 0))
_agg_blk = pl.BlockSpec((NC, BLK, C), lambda i: (0, i, 0))
_bias_blk = pl.BlockSpec((1, C), lambda i: (0, 0))
_out_t = jax.ShapeDtypeStruct((N, C), jnp.float32)


@jax.jit
def _pre(x, W1, dp):
    return pl.pallas_call(
        _pre_body, grid=(N // BLK,),
        in_specs=[_row_blk, _full_w, _dp_blk],
        out_specs=_row_blk, out_shape=_out_t,
    )(x, W1, dp)


@jax.jit
def _mid(agg, m, dp, b, W2):
    return pl.pallas_call(
        _mid_body, grid=(N // BLK,),
        in_specs=[_agg_blk, _row_blk, _dp_blk, _bias_blk, _full_w],
        out_specs=_row_blk, out_shape=_out_t,
    )(agg, m, dp, b, W2)


@jax.jit
def _post(agg, m, dp, b):
    return pl.pallas_call(
        _post_body, grid=(N // BLK,),
        in_specs=[_agg_blk, _row_blk, _dp_blk, _bias_blk],
        out_specs=_row_blk, out_shape=_out_t,
    )(agg, m, dp, b)


def kernel(x, edge_index, W1, b1, W2, b2):
    src2d = edge_index[0].astype(jnp.int32).reshape(E // CHUNK, CHUNK)
    dst2d = edge_index[1].astype(jnp.int32).reshape(E // CHUNK, CHUNK)
    b1r = b1.reshape(1, C)
    b2r = b2.reshape(1, C)

    dp = _deg(dst2d)
    m1 = _pre(x, W1, dp)
    agg1 = _agg(m1, src2d, dst2d)
    m2 = _mid(agg1, m1, dp, b1r, W2)
    agg2 = _agg(m2, src2d, dst2d)
    return _post(agg2, m2, dp, b2r)
